# baseline (speedup 1.0000x reference)
import functools

import jax
import jax.numpy as jnp
from jax import lax
from jax.experimental import pallas as pl
from jax.experimental.pallas import tpu as pltpu
from jax.experimental.pallas import tpu_sc as plsc

N = 10000
E = 320000
D = 128
DH = 64
NC = 2
NS = 16
EPT = E // NS
G = 80
NCH = EPT // G
RPS = N // NS
ZB = 25
ND = 10240
DPS = ND // NS
DROWS = NCH // NC

BLK = 2000


def _sc_segment_sum(p_split, src_t, dst_t, compute_deg, NBUF):
    mesh = plsc.VectorSubcoreMesh(core_axis_name="c", subcore_axis_name="s")

    out_type = [jax.ShapeDtypeStruct((NC, N, DH), jnp.float32)]
    scratch = [
        pltpu.VMEM((NCH, G), jnp.int32),
        pltpu.VMEM((NCH, G), jnp.int32),
    ] + [pltpu.VMEM((G, DH), jnp.float32) for _ in range(NBUF)] + [
        pltpu.VMEM((ZB, DH), jnp.float32),
        pltpu.VMEM_SHARED((N, DH), jnp.float32),
    ] + [pltpu.SemaphoreType.DMA for _ in range(NBUF)]
    if compute_deg:
        out_type.append(jax.ShapeDtypeStruct((NC, ND), jnp.float32))
        scratch += [
            pltpu.VMEM((ND,), jnp.float32),
            pltpu.VMEM((DPS,), jnp.float32),
            pltpu.VMEM_SHARED((NS, ND), jnp.float32),
        ]

    @functools.partial(
        pl.kernel,
        mesh=mesh,
        compiler_params=pltpu.CompilerParams(
            use_tc_tiling_on_sc=False, needs_layout_passes=False),
        out_type=tuple(out_type),
        scratch_types=scratch,
    )
    def k(p_hbm, src_hbm, dst_hbm, *refs):
        if compute_deg:
            acc_hbm, deg_hbm = refs[0], refs[1]
            rest = refs[2:]
        else:
            acc_hbm = refs[0]
            rest = refs[1:]
        src_v, dst_v = rest[0], rest[1]
        bufs = rest[2:2 + NBUF]
        buf_v, acc_sh = rest[2 + NBUF], rest[3 + NBUF]
        sems = rest[4 + NBUF:4 + 2 * NBUF]
        if compute_deg:
            degv, dsum, dstage = rest[4 + 2 * NBUF:]

        c = lax.axis_index("c")
        s = lax.axis_index("s")

        pltpu.sync_copy(src_hbm.at[c, s], src_v)
        pltpu.sync_copy(dst_hbm.at[s], dst_v)

        zeros16 = jnp.zeros((16,), jnp.float32)

        def zrow(i, carry):
            for j in range(DH // 16):
                buf_v[i, pl.ds(j * 16, 16)] = zeros16
            return carry

        lax.fori_loop(0, ZB, zrow, 0)

        def zslab(i, carry):
            pltpu.sync_copy(buf_v, acc_sh.at[pl.ds(s * RPS + i * ZB, ZB)])
            return carry

        lax.fori_loop(0, RPS // ZB, zslab, 0)

        if compute_deg:
            def zdeg(i, carry):
                degv[pl.ds(i * 16, 16)] = zeros16
                return carry

            lax.fori_loop(0, ND // 16, zdeg, 0)

        plsc.subcore_barrier()

        def start_g(j, buf, sem):
            pltpu.async_copy(p_hbm.at[src_v.at[j]], buf, sem)

        def wait_g(j, buf, sem):
            pltpu.make_async_copy(p_hbm.at[src_v.at[j]], buf, sem).wait()

        def start_s(j, buf, sem):
            pltpu.async_copy(buf, acc_sh.at[dst_v.at[j]], sem, add=True)

        def wait_s(j, buf, sem):
            pltpu.make_async_copy(buf, acc_sh.at[dst_v.at[j]], sem).wait()

        ones16 = jnp.ones((16,), jnp.float32)

        def deg_row(t):
            row = c * DROWS + t
            for j in range(G // 16):
                idx = dst_v[row, pl.ds(j * 16, 16)]
                plsc.addupdate_scatter(degv, [idx], ones16)

        for k in range(NBUF):
            start_g(k, bufs[k], sems[k])

        QT = (NCH - 2 * NBUF) // NBUF + 1
        JE = QT * NBUF

        def quad(t, carry):
            for k in range(NBUF):
                j = NBUF * t + k
                wait_g(j, bufs[k], sems[k])
                start_s(j, bufs[k], sems[k])
                if compute_deg and k in (0, 2):
                    deg_row(2 * t + k // 2)
                wait_s(j, bufs[k], sems[k])
                start_g(j + NBUF, bufs[k], sems[k])
            return carry

        lax.fori_loop(0, QT, quad, 0)

        for j in range(JE, NCH):
            k = j % NBUF
            wait_g(j, bufs[k], sems[k])
            start_s(j, bufs[k], sems[k])
            wait_s(j, bufs[k], sems[k])
            if j + NBUF < NCH:
                start_g(j + NBUF, bufs[k], sems[k])
        if compute_deg:
            for r in range(2 * QT, DROWS):
                deg_row(r)
        plsc.subcore_barrier()

        def wslab(i, carry):
            pltpu.sync_copy(acc_sh.at[pl.ds(s * RPS + i * ZB, ZB)], buf_v)
            pltpu.sync_copy(buf_v, acc_hbm.at[c, pl.ds(s * RPS + i * ZB, ZB)])
            return carry

        lax.fori_loop(0, RPS // ZB, wslab, 0)

        if compute_deg:
            pltpu.sync_copy(degv, dstage.at[s])
            plsc.subcore_barrier()

            def dzero(i, carry):
                dsum[pl.ds(i * 16, 16)] = zeros16
                return carry

            lax.fori_loop(0, DPS // 16, dzero, 0)

            def dred(r, carry):
                pltpu.sync_copy(dstage.at[r, pl.ds(s * DPS, DPS)], degv.at[pl.ds(0, DPS)])
                for i in range(DPS // 16):
                    sl = pl.ds(i * 16, 16)
                    dsum[sl] = dsum[sl] + degv[sl]
                return carry

            lax.fori_loop(0, NS, dred, 0)
            pltpu.sync_copy(dsum, deg_hbm.at[c, pl.ds(s * DPS, DPS)])

    return k(p_split, src_t, dst_t)


def _split_halves(p, out_ref):
    out_ref[:, 0, :] = p[:, 0:DH]
    out_ref[:, 1, :] = p[:, DH:D]


def _tc_project(x, Wl):
    def body(x_ref, wl_ref, out_ref):
        p = jnp.dot(x_ref[...], wl_ref[...], preferred_element_type=jnp.float32)
        _split_halves(p, out_ref)

    return pl.pallas_call(
        body,
        grid=(N // BLK,),
        in_specs=[
            pl.BlockSpec((BLK, D), lambda i: (i, 0)),
            pl.BlockSpec((D, D), lambda i: (0, 0)),
        ],
        out_specs=pl.BlockSpec((BLK, 2, DH), lambda i: (i, 0, 0)),
        out_shape=jax.ShapeDtypeStruct((N, 2, DH), jnp.float32),
    )(x, Wl)


def _agg_from_acc(acc_ref, d0_ref, d1_ref):
    deg = jnp.clip(d0_ref[...] + d1_ref[...], 1.0, None)
    agg = jnp.concatenate([acc_ref[0], acc_ref[1]], axis=1) / deg
    return agg


def _tc_mid(acc, d0, d1, x, Wr1, b1, Wl2, Wr2, b2):
    def body(acc_ref, d0_ref, d1_ref, x_ref, wr1_ref, b1_ref, wl2_ref,
             wr2_ref, b2_ref, p2_ref, r2_ref):
        agg = _agg_from_acc(acc_ref, d0_ref, d1_ref)
        h = jnp.maximum(
            agg + jnp.dot(x_ref[...], wr1_ref[...],
                          preferred_element_type=jnp.float32) + b1_ref[...],
            0.0)
        p2 = jnp.dot(h, wl2_ref[...], preferred_element_type=jnp.float32)
        _split_halves(p2, p2_ref)
        r2_ref[...] = jnp.dot(h, wr2_ref[...],
                              preferred_element_type=jnp.float32) + b2_ref[...]

    return pl.pallas_call(
        body,
        grid=(N // BLK,),
        in_specs=[
            pl.BlockSpec((NC, BLK, DH), lambda i: (0, i, 0)),
            pl.BlockSpec((BLK, 1), lambda i: (i, 0)),
            pl.BlockSpec((BLK, 1), lambda i: (i, 0)),
            pl.BlockSpec((BLK, D), lambda i: (i, 0)),
            pl.BlockSpec((D, D), lambda i: (0, 0)),
            pl.BlockSpec((1, D), lambda i: (0, 0)),
            pl.BlockSpec((D, D), lambda i: (0, 0)),
            pl.BlockSpec((D, D), lambda i: (0, 0)),
            pl.BlockSpec((1, D), lambda i: (0, 0)),
        ],
        out_specs=[
            pl.BlockSpec((BLK, 2, DH), lambda i: (i, 0, 0)),
            pl.BlockSpec((BLK, D), lambda i: (i, 0)),
        ],
        out_shape=[
            jax.ShapeDtypeStruct((N, 2, DH), jnp.float32),
            jax.ShapeDtypeStruct((N, D), jnp.float32),
        ],
    )(acc, d0, d1, x, Wr1, b1, Wl2, Wr2, b2)


def _tc_final(acc, d0, d1, r2):
    def body(acc_ref, d0_ref, d1_ref, r2_ref, out_ref):
        t = _agg_from_acc(acc_ref, d0_ref, d1_ref) + r2_ref[...]
        m = jnp.max(t, axis=-1, keepdims=True)
        lse = m + jnp.log(jnp.sum(jnp.exp(t - m), axis=-1, keepdims=True))
        out_ref[...] = t - lse

    return pl.pallas_call(
        body,
        grid=(N // BLK,),
        in_specs=[
            pl.BlockSpec((NC, BLK, DH), lambda i: (0, i, 0)),
            pl.BlockSpec((BLK, 1), lambda i: (i, 0)),
            pl.BlockSpec((BLK, 1), lambda i: (i, 0)),
            pl.BlockSpec((BLK, D), lambda i: (i, 0)),
        ],
        out_specs=pl.BlockSpec((BLK, D), lambda i: (i, 0)),
        out_shape=jax.ShapeDtypeStruct((N, D), jnp.float32),
    )(acc, d0, d1, r2)


def kernel(x, edge_index, Wl1, Wr1, b1, Wl2, Wr2, b2):
    src = edge_index[0].astype(jnp.int32).reshape(NS, NCH, G)
    src = jnp.stack([2 * src, 2 * src + 1])
    dst = edge_index[1].astype(jnp.int32).reshape(NS, NCH, G)
    b1r = b1.reshape(1, D)
    b2r = b2.reshape(1, D)

    p1 = _tc_project(x, Wl1).reshape(2 * N, DH)
    acc1, deg = _sc_segment_sum(p1, src, dst, compute_deg=True, NBUF=5)
    d0 = deg[0, :N].reshape(N, 1)
    d1 = deg[1, :N].reshape(N, 1)
    p2, r2 = _tc_mid(acc1, d0, d1, x, Wr1, b1r, Wl2, Wr2, b2r)
    (acc2,) = _sc_segment_sum(p2.reshape(2 * N, DH), src, dst,
                              compute_deg=False, NBUF=9)
    return _tc_final(acc2, d0, d1, r2)


# speedup vs baseline: 1.2223x; 1.2223x over previous
import functools

import jax
import jax.numpy as jnp
from jax import lax
from jax.experimental import pallas as pl
from jax.experimental.pallas import tpu as pltpu
from jax.experimental.pallas import tpu_sc as plsc

N = 10000
E = 320000
D = 128
DH = 64
NC = 2
NS = 16
EPT = E // NS
G = 80
NCH = EPT // G
RPS = N // NS
ZB = 125
ND = 10240
DPS = ND // NS
DROWS = NCH // NC
NBUF = 4

BLK = 2000


def _sc_segment_sum(p_split, src_t, dst_t, compute_deg):
    mesh = plsc.VectorSubcoreMesh(core_axis_name="c", subcore_axis_name="s")

    out_type = [jax.ShapeDtypeStruct((N, D), jnp.float32)]
    scratch = [
        pltpu.VMEM((NCH, G), jnp.int32),
        pltpu.VMEM((NCH, G), jnp.int32),
    ] + [pltpu.VMEM((G, DH), jnp.float32) for _ in range(NBUF)] + [
        pltpu.VMEM((ZB, DH), jnp.float32),
        pltpu.VMEM_SHARED((N, DH), jnp.float32),
    ] + [pltpu.SemaphoreType.DMA for _ in range(NBUF)]
    if compute_deg:
        out_type.append(jax.ShapeDtypeStruct((NC, ND), jnp.float32))
        scratch += [
            pltpu.VMEM((ND,), jnp.float32),
            pltpu.VMEM((DPS,), jnp.float32),
            pltpu.VMEM_SHARED((NS, ND), jnp.float32),
        ]

    @functools.partial(
        pl.kernel,
        mesh=mesh,
        compiler_params=pltpu.CompilerParams(
            use_tc_tiling_on_sc=False, needs_layout_passes=False),
        out_type=tuple(out_type),
        scratch_types=scratch,
    )
    def k(p_hbm, src_hbm, dst_hbm, *refs):
        if compute_deg:
            acc_hbm, deg_hbm = refs[0], refs[1]
            rest = refs[2:]
        else:
            acc_hbm = refs[0]
            rest = refs[1:]
        src_v, dst_v = rest[0], rest[1]
        bufs = rest[2:2 + NBUF]
        buf_v, acc_sh = rest[2 + NBUF], rest[3 + NBUF]
        sems = rest[4 + NBUF:4 + 2 * NBUF]
        if compute_deg:
            degv, dsum, dstage = rest[4 + 2 * NBUF:]

        c = lax.axis_index("c")
        s = lax.axis_index("s")

        pltpu.sync_copy(src_hbm.at[c, s], src_v)
        pltpu.sync_copy(dst_hbm.at[s], dst_v)

        zeros16 = jnp.zeros((16,), jnp.float32)

        def zrow(i, carry):
            for j in range(DH // 16):
                buf_v[i, pl.ds(j * 16, 16)] = zeros16
            return carry

        lax.fori_loop(0, ZB, zrow, 0)

        def zslab(i, carry):
            pltpu.sync_copy(buf_v, acc_sh.at[pl.ds(s * RPS + i * ZB, ZB)])
            return carry

        lax.fori_loop(0, RPS // ZB, zslab, 0)

        if compute_deg:
            def zdeg(i, carry):
                degv[pl.ds(i * 16, 16)] = zeros16
                return carry

            lax.fori_loop(0, ND // 16, zdeg, 0)

        plsc.subcore_barrier()

        def start_g(j, buf, sem):
            pltpu.async_copy(p_hbm.at[src_v.at[j]], buf, sem)

        def wait_g(j, buf, sem):
            pltpu.make_async_copy(p_hbm.at[src_v.at[j]], buf, sem).wait()

        def start_s(j, buf, sem):
            pltpu.async_copy(buf, acc_sh.at[dst_v.at[j]], sem, add=True)

        def wait_s(j, buf, sem):
            pltpu.make_async_copy(buf, acc_sh.at[dst_v.at[j]], sem).wait()

        ones16 = jnp.ones((16,), jnp.float32)

        def deg_row(t):
            row = c * DROWS + t
            for j in range(G // 16):
                idx = dst_v[row, pl.ds(j * 16, 16)]
                plsc.addupdate_scatter(degv, [idx], ones16)

        for k in range(NBUF):
            start_g(k, bufs[k], sems[k])

        QT = (NCH - 2 * NBUF) // NBUF + 1
        JE = QT * NBUF

        def quad(t, carry):
            for k in range(NBUF):
                j = NBUF * t + k
                wait_g(j, bufs[k], sems[k])
                start_s(j, bufs[k], sems[k])
                if compute_deg and k in (0, 2):
                    deg_row(2 * t + k // 2)
                wait_s(j, bufs[k], sems[k])
                start_g(j + NBUF, bufs[k], sems[k])
            return carry

        lax.fori_loop(0, QT, quad, 0)

        for j in range(JE, NCH):
            k = j % NBUF
            wait_g(j, bufs[k], sems[k])
            start_s(j, bufs[k], sems[k])
            wait_s(j, bufs[k], sems[k])
            if j + NBUF < NCH:
                start_g(j + NBUF, bufs[k], sems[k])
        if compute_deg:
            for r in range(2 * QT, DROWS):
                deg_row(r)
        plsc.subcore_barrier()

        def wslab(i, carry):
            rows = pl.ds(s * RPS + i * ZB, ZB)
            pltpu.sync_copy(acc_sh.at[rows], buf_v)
            pltpu.sync_copy(buf_v, acc_hbm.at[rows, pl.ds(c * DH, DH)])
            return carry

        lax.fori_loop(0, RPS // ZB, wslab, 0)

        if compute_deg:
            pltpu.sync_copy(degv, dstage.at[s])
            plsc.subcore_barrier()

            def dzero(i, carry):
                dsum[pl.ds(i * 16, 16)] = zeros16
                return carry

            lax.fori_loop(0, DPS // 16, dzero, 0)

            def dred(r, carry):
                pltpu.sync_copy(dstage.at[r, pl.ds(s * DPS, DPS)], degv.at[pl.ds(0, DPS)])
                for i in range(DPS // 16):
                    sl = pl.ds(i * 16, 16)
                    dsum[sl] = dsum[sl] + degv[sl]
                return carry

            lax.fori_loop(0, NS, dred, 0)
            pltpu.sync_copy(dsum, deg_hbm.at[c, pl.ds(s * DPS, DPS)])

    return k(p_split, src_t, dst_t)


def _tc_project(x, Wl):
    def body(x_ref, wl_ref, out_ref):
        out_ref[...] = jnp.dot(x_ref[...], wl_ref[...],
                               preferred_element_type=jnp.float32)

    return pl.pallas_call(
        body,
        grid=(N // BLK,),
        in_specs=[
            pl.BlockSpec((BLK, D), lambda i: (i, 0)),
            pl.BlockSpec((D, D), lambda i: (0, 0)),
        ],
        out_specs=pl.BlockSpec((BLK, D), lambda i: (i, 0)),
        out_shape=jax.ShapeDtypeStruct((N, D), jnp.float32),
    )(x, Wl)


def _agg_from_acc(acc_ref, deg_ref):
    return acc_ref[...] / jnp.clip(deg_ref[...], 1.0, None)


def _tc_mid(acc, deg, x, Wr1, b1, Wl2, Wr2, b2):
    def body(acc_ref, deg_ref, x_ref, wr1_ref, b1_ref, wl2_ref,
             wr2_ref, b2_ref, p2_ref, r2_ref):
        agg = _agg_from_acc(acc_ref, deg_ref)
        h = jnp.maximum(
            agg + jnp.dot(x_ref[...], wr1_ref[...],
                          preferred_element_type=jnp.float32) + b1_ref[...],
            0.0)
        p2_ref[...] = jnp.dot(h, wl2_ref[...],
                              preferred_element_type=jnp.float32)
        r2_ref[...] = jnp.dot(h, wr2_ref[...],
                              preferred_element_type=jnp.float32) + b2_ref[...]

    return pl.pallas_call(
        body,
        grid=(N // BLK,),
        in_specs=[
            pl.BlockSpec((BLK, D), lambda i: (i, 0)),
            pl.BlockSpec((BLK, 1), lambda i: (i, 0)),
            pl.BlockSpec((BLK, D), lambda i: (i, 0)),
            pl.BlockSpec((D, D), lambda i: (0, 0)),
            pl.BlockSpec((1, D), lambda i: (0, 0)),
            pl.BlockSpec((D, D), lambda i: (0, 0)),
            pl.BlockSpec((D, D), lambda i: (0, 0)),
            pl.BlockSpec((1, D), lambda i: (0, 0)),
        ],
        out_specs=[
            pl.BlockSpec((BLK, D), lambda i: (i, 0)),
            pl.BlockSpec((BLK, D), lambda i: (i, 0)),
        ],
        out_shape=[
            jax.ShapeDtypeStruct((N, D), jnp.float32),
            jax.ShapeDtypeStruct((N, D), jnp.float32),
        ],
    )(acc, deg, x, Wr1, b1, Wl2, Wr2, b2)


def _tc_final(acc, deg, r2):
    def body(acc_ref, deg_ref, r2_ref, out_ref):
        t = _agg_from_acc(acc_ref, deg_ref) + r2_ref[...]
        m = jnp.max(t, axis=-1, keepdims=True)
        lse = m + jnp.log(jnp.sum(jnp.exp(t - m), axis=-1, keepdims=True))
        out_ref[...] = t - lse

    return pl.pallas_call(
        body,
        grid=(N // BLK,),
        in_specs=[
            pl.BlockSpec((BLK, D), lambda i: (i, 0)),
            pl.BlockSpec((BLK, 1), lambda i: (i, 0)),
            pl.BlockSpec((BLK, D), lambda i: (i, 0)),
        ],
        out_specs=pl.BlockSpec((BLK, D), lambda i: (i, 0)),
        out_shape=jax.ShapeDtypeStruct((N, D), jnp.float32),
    )(acc, deg, r2)


def kernel(x, edge_index, Wl1, Wr1, b1, Wl2, Wr2, b2):
    src = edge_index[0].astype(jnp.int32).reshape(NS, NCH, G)
    src = jnp.stack([2 * src, 2 * src + 1])
    dst = edge_index[1].astype(jnp.int32).reshape(NS, NCH, G)
    b1r = b1.reshape(1, D)
    b2r = b2.reshape(1, D)

    p1 = _tc_project(x, Wl1).reshape(2 * N, DH)
    acc1, deg = _sc_segment_sum(p1, src, dst, compute_deg=True)
    deg = (deg[0, :N] + deg[1, :N]).reshape(N, 1)
    p2, r2 = _tc_mid(acc1, deg, x, Wr1, b1r, Wl2, Wr2, b2r)
    (acc2,) = _sc_segment_sum(p2.reshape(2 * N, DH), src, dst,
                              compute_deg=False)
    return _tc_final(acc2, deg, r2)


# speedup vs baseline: 1.2437x; 1.0175x over previous
import functools

import jax
import jax.numpy as jnp
from jax import lax
from jax.experimental import pallas as pl
from jax.experimental.pallas import tpu as pltpu
from jax.experimental.pallas import tpu_sc as plsc

N = 10000
E = 320000
D = 128
DH = 64
NC = 2
NS = 16
EPT = E // NS
G = 80
NCH = EPT // G
RPS = N // NS
ZB = 125
ND = 10240
DPS = ND // NS
DROWS = NCH // NC
NBUF = 4

BLK = 2000


def _sc_segment_sum(p_split, src_t, dst_t, compute_deg):
    mesh = plsc.VectorSubcoreMesh(core_axis_name="c", subcore_axis_name="s")

    out_type = [jax.ShapeDtypeStruct((N, D), jnp.float32)]
    scratch = [
        pltpu.VMEM((NCH, G), jnp.int32),
        pltpu.VMEM((NCH, G), jnp.int32),
    ] + [pltpu.VMEM((G, DH), jnp.float32) for _ in range(NBUF)] + [
        pltpu.VMEM((ZB, DH), jnp.float32),
        pltpu.VMEM_SHARED((N, DH), jnp.float32),
    ] + [pltpu.SemaphoreType.DMA for _ in range(NBUF)]
    if compute_deg:
        out_type.append(jax.ShapeDtypeStruct((NC, ND), jnp.float32))
        scratch += [
            pltpu.VMEM((ND,), jnp.float32),
            pltpu.VMEM((DPS,), jnp.float32),
            pltpu.VMEM_SHARED((NS, ND), jnp.float32),
        ]

    @functools.partial(
        pl.kernel,
        mesh=mesh,
        compiler_params=pltpu.CompilerParams(
            use_tc_tiling_on_sc=False, needs_layout_passes=False),
        out_type=tuple(out_type),
        scratch_types=scratch,
    )
    def k(p_hbm, src_hbm, dst_hbm, *refs):
        if compute_deg:
            acc_hbm, deg_hbm = refs[0], refs[1]
            rest = refs[2:]
        else:
            acc_hbm = refs[0]
            rest = refs[1:]
        src_v, dst_v = rest[0], rest[1]
        bufs = rest[2:2 + NBUF]
        buf_v, acc_sh = rest[2 + NBUF], rest[3 + NBUF]
        sems = rest[4 + NBUF:4 + 2 * NBUF]
        if compute_deg:
            degv, dsum, dstage = rest[4 + 2 * NBUF:]

        c = lax.axis_index("c")
        s = lax.axis_index("s")

        pltpu.sync_copy(src_hbm.at[c, s], src_v)
        pltpu.sync_copy(dst_hbm.at[s], dst_v)

        zeros16 = jnp.zeros((16,), jnp.float32)

        def zrow(i, carry):
            for j in range(DH // 16):
                buf_v[i, pl.ds(j * 16, 16)] = zeros16
            return carry

        lax.fori_loop(0, ZB, zrow, 0)

        def zslab(i, carry):
            pltpu.sync_copy(buf_v, acc_sh.at[pl.ds(s * RPS + i * ZB, ZB)])
            return carry

        lax.fori_loop(0, RPS // ZB, zslab, 0)

        if compute_deg:
            def zdeg(i, carry):
                degv[pl.ds(i * 16, 16)] = zeros16
                return carry

            lax.fori_loop(0, ND // 16, zdeg, 0)

        plsc.subcore_barrier()

        def start_g(j, buf, sem):
            pltpu.async_copy(p_hbm.at[src_v.at[j]], buf, sem)

        def wait_g(j, buf, sem):
            pltpu.make_async_copy(p_hbm.at[src_v.at[j]], buf, sem).wait()

        def start_s(j, buf, sem):
            pltpu.async_copy(buf, acc_sh.at[dst_v.at[j]], sem, add=True)

        def wait_s(j, buf, sem):
            pltpu.make_async_copy(buf, acc_sh.at[dst_v.at[j]], sem).wait()

        ones16 = jnp.ones((16,), jnp.float32)

        def deg_row(t):
            row = c * DROWS + t
            for j in range(G // 16):
                idx = dst_v[row, pl.ds(j * 16, 16)]
                plsc.addupdate_scatter(degv, [idx], ones16)

        for k in range(NBUF):
            start_g(k, bufs[k], sems[k])

        QT = (NCH - 2 * NBUF) // NBUF + 1
        JE = QT * NBUF

        def quad(t, carry):
            for k in range(NBUF):
                j = NBUF * t + k
                wait_g(j, bufs[k], sems[k])
                start_s(j, bufs[k], sems[k])
                if compute_deg and k in (0, 2):
                    deg_row(2 * t + k // 2)
                wait_s(j, bufs[k], sems[k])
                start_g(j + NBUF, bufs[k], sems[k])
            return carry

        lax.fori_loop(0, QT, quad, 0)

        for j in range(JE, NCH):
            k = j % NBUF
            wait_g(j, bufs[k], sems[k])
            start_s(j, bufs[k], sems[k])
            wait_s(j, bufs[k], sems[k])
            if j + NBUF < NCH:
                start_g(j + NBUF, bufs[k], sems[k])
        if compute_deg:
            for r in range(2 * QT, DROWS):
                deg_row(r)
        plsc.subcore_barrier()

        def wslab(i, carry):
            rows = pl.ds(s * RPS + i * ZB, ZB)
            pltpu.sync_copy(acc_sh.at[rows], buf_v)
            pltpu.sync_copy(buf_v, acc_hbm.at[rows, pl.ds(c * DH, DH)])
            return carry

        lax.fori_loop(0, RPS // ZB, wslab, 0)

        if compute_deg:
            pltpu.sync_copy(degv, dstage.at[s])
            plsc.subcore_barrier()

            def dzero(i, carry):
                dsum[pl.ds(i * 16, 16)] = zeros16
                return carry

            lax.fori_loop(0, DPS // 16, dzero, 0)

            def dred(r, carry):
                pltpu.sync_copy(dstage.at[r, pl.ds(s * DPS, DPS)], degv.at[pl.ds(0, DPS)])
                for i in range(DPS // 16):
                    sl = pl.ds(i * 16, 16)
                    dsum[sl] = dsum[sl] + degv[sl]
                return carry

            lax.fori_loop(0, NS, dred, 0)
            pltpu.sync_copy(dsum, deg_hbm.at[c, pl.ds(s * DPS, DPS)])

    return k(p_split, src_t, dst_t)


EROWS = E // 128


def _tc_idx_prep(srcf):
    def body(s_ref, o_ref):
        s2 = s_ref[...] * 2
        o_ref[0] = s2
        o_ref[1] = s2 + 1

    return pl.pallas_call(
        body,
        in_specs=[pl.BlockSpec((EROWS, 128), lambda: (0, 0))],
        out_specs=pl.BlockSpec((NC, EROWS, 128), lambda: (0, 0, 0)),
        out_shape=jax.ShapeDtypeStruct((NC, EROWS, 128), jnp.int32),
    )(srcf)


def _tc_project(x, Wl):
    def body(x_ref, wl_ref, out_ref):
        out_ref[...] = jnp.dot(x_ref[...], wl_ref[...],
                               preferred_element_type=jnp.float32)

    return pl.pallas_call(
        body,
        grid=(N // BLK,),
        in_specs=[
            pl.BlockSpec((BLK, D), lambda i: (i, 0)),
            pl.BlockSpec((D, D), lambda i: (0, 0)),
        ],
        out_specs=pl.BlockSpec((BLK, D), lambda i: (i, 0)),
        out_shape=jax.ShapeDtypeStruct((N, D), jnp.float32),
    )(x, Wl)


def _agg_from_acc(acc_ref, deg_ref):
    return acc_ref[...] / jnp.clip(deg_ref[...], 1.0, None)


def _tc_mid(acc, deg, x, Wr1, b1, Wl2, Wr2, b2):
    def body(acc_ref, deg_ref, x_ref, wr1_ref, b1_ref, wl2_ref,
             wr2_ref, b2_ref, p2_ref, r2_ref):
        agg = _agg_from_acc(acc_ref, deg_ref)
        h = jnp.maximum(
            agg + jnp.dot(x_ref[...], wr1_ref[...],
                          preferred_element_type=jnp.float32) + b1_ref[...],
            0.0)
        p2_ref[...] = jnp.dot(h, wl2_ref[...],
                              preferred_element_type=jnp.float32)
        r2_ref[...] = jnp.dot(h, wr2_ref[...],
                              preferred_element_type=jnp.float32) + b2_ref[...]

    return pl.pallas_call(
        body,
        grid=(N // BLK,),
        in_specs=[
            pl.BlockSpec((BLK, D), lambda i: (i, 0)),
            pl.BlockSpec((BLK, 1), lambda i: (i, 0)),
            pl.BlockSpec((BLK, D), lambda i: (i, 0)),
            pl.BlockSpec((D, D), lambda i: (0, 0)),
            pl.BlockSpec((1, D), lambda i: (0, 0)),
            pl.BlockSpec((D, D), lambda i: (0, 0)),
            pl.BlockSpec((D, D), lambda i: (0, 0)),
            pl.BlockSpec((1, D), lambda i: (0, 0)),
        ],
        out_specs=[
            pl.BlockSpec((BLK, D), lambda i: (i, 0)),
            pl.BlockSpec((BLK, D), lambda i: (i, 0)),
        ],
        out_shape=[
            jax.ShapeDtypeStruct((N, D), jnp.float32),
            jax.ShapeDtypeStruct((N, D), jnp.float32),
        ],
    )(acc, deg, x, Wr1, b1, Wl2, Wr2, b2)


def _tc_final(acc, deg, r2):
    def body(acc_ref, deg_ref, r2_ref, out_ref):
        t = _agg_from_acc(acc_ref, deg_ref) + r2_ref[...]
        m = jnp.max(t, axis=-1, keepdims=True)
        lse = m + jnp.log(jnp.sum(jnp.exp(t - m), axis=-1, keepdims=True))
        out_ref[...] = t - lse

    return pl.pallas_call(
        body,
        grid=(N // BLK,),
        in_specs=[
            pl.BlockSpec((BLK, D), lambda i: (i, 0)),
            pl.BlockSpec((BLK, 1), lambda i: (i, 0)),
            pl.BlockSpec((BLK, D), lambda i: (i, 0)),
        ],
        out_specs=pl.BlockSpec((BLK, D), lambda i: (i, 0)),
        out_shape=jax.ShapeDtypeStruct((N, D), jnp.float32),
    )(acc, deg, r2)


def kernel(x, edge_index, Wl1, Wr1, b1, Wl2, Wr2, b2):
    srcf = edge_index[0].astype(jnp.int32).reshape(EROWS, 128)
    src = _tc_idx_prep(srcf).reshape(NC, NS, NCH, G)
    dst = edge_index[1].astype(jnp.int32).reshape(NS, NCH, G)
    b1r = b1.reshape(1, D)
    b2r = b2.reshape(1, D)

    p1 = _tc_project(x, Wl1).reshape(2 * N, DH)
    acc1, deg = _sc_segment_sum(p1, src, dst, compute_deg=True)
    deg = (deg[0, :N] + deg[1, :N]).reshape(N, 1)
    p2, r2 = _tc_mid(acc1, deg, x, Wr1, b1r, Wl2, Wr2, b2r)
    (acc2,) = _sc_segment_sum(p2.reshape(2 * N, DH), src, dst,
                              compute_deg=False)
    return _tc_final(acc2, deg, r2)
